# Initial kernel scaffold; baseline (speedup 1.0000x reference)
import jax
import jax.numpy as jnp
from jax.experimental import pallas as pl


def kernel(x, edge_index, batch, edge_attr, pos, params):
    raise NotImplementedError("write your pallas kernel here")


# speedup vs baseline: 64.7773x; 64.7773x over previous
import functools

import jax
import jax.numpy as jnp
from jax import lax
from jax.experimental import pallas as pl
from jax.experimental.pallas import tpu as pltpu
from jax.experimental.pallas import tpu_sc as plsc

N = 50000
E = 800000
D = 64
K1 = 25000
K2 = 12500
NP = 50176
NBLK = 8
BLK = NP // NBLK
SROW = NP // 128

_SC_NC = 2
_SC_NS = 16
_NW = _SC_NC * _SC_NS
_EW = E // _NW
_EB = 5000


def _segsum8_sc(vals, row, col, zeros):
    mesh = plsc.VectorSubcoreMesh(core_axis_name="c", subcore_axis_name="s")

    @functools.partial(
        pl.kernel, mesh=mesh,
        compiler_params=pltpu.CompilerParams(use_tc_tiling_on_sc=False),
        out_type=jax.ShapeDtypeStruct((_SC_NC, NP, 8), jnp.float32),
        scratch_types=[
            pltpu.VMEM((_EB,), jnp.int32),
            pltpu.VMEM((_EB,), jnp.int32),
            pltpu.VMEM((_EB, 8), jnp.float32),
            pltpu.VMEM_SHARED((NP, 8), jnp.float32),
            pltpu.VMEM_SHARED((NP, 8), jnp.float32),
            pltpu.SemaphoreType.DMA,
        ])
    def k(vals_hbm, row_hbm, col_hbm, zeros_hbm, out_hbm, ridx, cidx, msg,
          val_sh, acc_sh, sem):
        c = lax.axis_index("c")
        s = lax.axis_index("s")
        wid = s * _SC_NC + c
        rows_per = NP // _SC_NS
        rbase = s * rows_per
        pltpu.sync_copy(vals_hbm.at[pl.ds(rbase, rows_per)],
                        val_sh.at[pl.ds(rbase, rows_per)])
        pltpu.sync_copy(zeros_hbm.at[pl.ds(rbase, rows_per)],
                        acc_sh.at[pl.ds(rbase, rows_per)])
        plsc.subcore_barrier()
        ebase = wid * _EW

        def win(j, carry):
            off = ebase + j * _EB
            pltpu.sync_copy(row_hbm.at[pl.ds(off, _EB)], ridx)
            pltpu.sync_copy(col_hbm.at[pl.ds(off, _EB)], cidx)
            pltpu.sync_copy(val_sh.at[ridx], msg)
            pltpu.sync_copy(msg, acc_sh.at[cidx], add=True)
            return carry

        lax.fori_loop(0, _EW // _EB, win, 0)
        plsc.subcore_barrier()
        pltpu.sync_copy(acc_sh.at[pl.ds(rbase, rows_per)],
                        out_hbm.at[c].at[pl.ds(rbase, rows_per)])

    return k(vals, row, col, zeros)


def _mm_kernel(x_ref, w_ref, o_ref):
    o_ref[...] = jnp.dot(x_ref[...], w_ref[...],
                         preferred_element_type=jnp.float32)


def _project(x, w):
    din, dout = w.shape
    return pl.pallas_call(
        _mm_kernel,
        grid=(NBLK,),
        in_specs=[pl.BlockSpec((BLK, din), lambda i: (i, 0)),
                  pl.BlockSpec((din, dout), lambda i: (0, 0))],
        out_specs=pl.BlockSpec((BLK, dout), lambda i: (i, 0)),
        out_shape=jax.ShapeDtypeStruct((NP, dout), jnp.float32),
    )(x, w)


def _gin_mlp_kernel(eps, y_ref, agg_ref, ba_ref, wb_ref, bb_ref, pv_ref,
                    h_ref, r_ref):
    z = eps * y_ref[...] + agg_ref[0] + agg_ref[1] + ba_ref[...]
    h = jnp.dot(jnp.maximum(z, 0.0), wb_ref[...],
                preferred_element_type=jnp.float32) + bb_ref[...]
    h_ref[...] = h
    r_ref[...] = jnp.dot(h, pv_ref[...], preferred_element_type=jnp.float32)


def _gin_mlp(eps, y, agg, ba, wb, bb, pvec):
    dout = wb.shape[1]
    return pl.pallas_call(
        functools.partial(_gin_mlp_kernel, eps),
        grid=(NBLK,),
        in_specs=[pl.BlockSpec((BLK, 8), lambda i: (i, 0)),
                  pl.BlockSpec((2, BLK, 8), lambda i: (0, i, 0)),
                  pl.BlockSpec((1, 8), lambda i: (0, 0)),
                  pl.BlockSpec((8, dout), lambda i: (0, 0)),
                  pl.BlockSpec((1, dout), lambda i: (0, 0)),
                  pl.BlockSpec((dout, 1), lambda i: (0, 0))],
        out_specs=[pl.BlockSpec((BLK, dout), lambda i: (i, 0)),
                   pl.BlockSpec((BLK, 1), lambda i: (i, 0))],
        out_shape=[jax.ShapeDtypeStruct((NP, dout), jnp.float32),
                   jax.ShapeDtypeStruct((NP, 1), jnp.float32)],
    )(y, agg, ba, wb, bb, pvec)


def _count_ge(bits, within, cand):
    return jnp.sum(jnp.where(within & (bits >= cand), 1, 0).astype(jnp.int32))


def _value_bitsearch(bits, within, k):
    def body(i, p):
        cand = p | (1 << (30 - i))
        return jnp.where(_count_ge(bits, within, cand) >= k, cand, p)
    return lax.fori_loop(0, 31, body, jnp.int32(0))


def _index_bitsearch(eq, idx, need):
    def body(i, q):
        cand = q | (1 << (15 - i))
        cnt = jnp.sum(jnp.where(eq & (idx < cand), 1, 0).astype(jnp.int32))
        return jnp.where(cnt < need, cand, q)
    return lax.fori_loop(0, 16, body, jnp.int32(0))


def _flat_idx():
    r = lax.broadcasted_iota(jnp.int32, (SROW, 128), 0)
    l = lax.broadcasted_iota(jnp.int32, (SROW, 128), 1)
    return r * 128 + l


def _select1_kernel(s_ref, g_ref, m_ref):
    s = s_ref[...]
    bits = jax.lax.bitcast_convert_type(s, jnp.int32)
    idx = _flat_idx()
    within = idx < N
    t = _value_bitsearch(bits, within, K1)
    gt = within & (bits > t)
    eq = within & (bits == t)
    need = K1 - jnp.sum(gt.astype(jnp.int32))
    q = _index_bitsearch(eq, idx, need)
    sel = gt | (eq & (idx <= q))
    m_ref[...] = sel.astype(jnp.float32)
    g_ref[...] = jnp.where(sel, s, 0.0)


def _select1(s1r):
    return pl.pallas_call(
        _select1_kernel,
        in_specs=[pl.BlockSpec((SROW, 128), lambda: (0, 0))],
        out_specs=[pl.BlockSpec((SROW, 128), lambda: (0, 0)),
                   pl.BlockSpec((SROW, 128), lambda: (0, 0))],
        out_shape=[jax.ShapeDtypeStruct((SROW, 128), jnp.float32),
                   jax.ShapeDtypeStruct((SROW, 128), jnp.float32)],
    )(s1r)


def _select2_kernel(s2_ref, s1_ref, m1_ref, g_ref, m_ref):
    s2 = s2_ref[...]
    m1 = m1_ref[...] > 0.0
    bits2 = jnp.where(m1, jax.lax.bitcast_convert_type(s2, jnp.int32), -1)
    bits1 = jax.lax.bitcast_convert_type(s1_ref[...], jnp.int32)
    idx = _flat_idx()
    allm = jnp.ones_like(m1)
    t2 = _value_bitsearch(bits2, allm, K2)
    gt2 = bits2 > t2
    eq2 = bits2 == t2
    need2 = K2 - jnp.sum(gt2.astype(jnp.int32))
    t1 = _value_bitsearch(bits1, eq2, need2)
    gt1 = eq2 & (bits1 > t1)
    eq21 = eq2 & (bits1 == t1)
    need1 = need2 - jnp.sum(gt1.astype(jnp.int32))
    q = _index_bitsearch(eq21, idx, need1)
    sel = gt2 | gt1 | (eq21 & (idx <= q))
    m_ref[...] = sel.astype(jnp.float32)
    g_ref[...] = jnp.where(sel, s2, 0.0)


def _select2(s2r, s1r, m1r):
    return pl.pallas_call(
        _select2_kernel,
        in_specs=[pl.BlockSpec((SROW, 128), lambda: (0, 0))] * 3,
        out_specs=[pl.BlockSpec((SROW, 128), lambda: (0, 0)),
                   pl.BlockSpec((SROW, 128), lambda: (0, 0))],
        out_shape=[jax.ShapeDtypeStruct((SROW, 128), jnp.float32),
                   jax.ShapeDtypeStruct((SROW, 128), jnp.float32)],
    )(s2r, s1r, m1r)


def _pool_project_kernel(kdiv, h_ref, g_ref, m_ref, w_ref, y_ref, x_ref,
                         accmax, accsum):
    i = pl.program_id(0)
    h = h_ref[...]
    g = g_ref[...]
    m = m_ref[...] > 0.0
    v = h * g
    y_ref[...] = jnp.dot(v, w_ref[...], preferred_element_type=jnp.float32)
    vmax = jnp.max(jnp.where(m, v, -jnp.inf), axis=0, keepdims=True)
    vsum = jnp.sum(v, axis=0, keepdims=True)

    @pl.when(i == 0)
    def _():
        accmax[...] = vmax
        accsum[...] = vsum

    @pl.when(i > 0)
    def _():
        accmax[...] = jnp.maximum(accmax[...], vmax)
        accsum[...] = accsum[...] + vsum

    @pl.when(i == NBLK - 1)
    def _():
        x_ref[...] = jnp.concatenate(
            [accmax[...], accsum[...] * kdiv], axis=1)


def _pool_project(h, gcol, mcol, w, kdiv):
    dh = h.shape[1]
    dw = w.shape[1]
    return pl.pallas_call(
        functools.partial(_pool_project_kernel, kdiv),
        grid=(NBLK,),
        in_specs=[pl.BlockSpec((BLK, dh), lambda i: (i, 0)),
                  pl.BlockSpec((BLK, 1), lambda i: (i, 0)),
                  pl.BlockSpec((BLK, 1), lambda i: (i, 0)),
                  pl.BlockSpec((dh, dw), lambda i: (0, 0))],
        out_specs=[pl.BlockSpec((BLK, dw), lambda i: (i, 0)),
                   pl.BlockSpec((1, 2 * dh), lambda i: (0, 0))],
        out_shape=[jax.ShapeDtypeStruct((NP, dw), jnp.float32),
                   jax.ShapeDtypeStruct((1, 2 * dh), jnp.float32)],
        scratch_shapes=[pltpu.VMEM((1, dh), jnp.float32),
                        pltpu.VMEM((1, dh), jnp.float32)],
    )(h, gcol, mcol, w)


def _tail_kernel(h_ref, g_ref, m_ref, x1_ref, w1_ref, w2_ref, w3_ref,
                 fcb_ref, hw_ref, hb_ref, ow_ref, ob_ref,
                 o1_ref, o2_ref, o3_ref, accmax, accsum):
    i = pl.program_id(0)
    h = h_ref[...]
    g = g_ref[...]
    m = m_ref[...] > 0.0
    v = h * g
    vmax = jnp.max(jnp.where(m, v, -jnp.inf), axis=0, keepdims=True)
    vsum = jnp.sum(v, axis=0, keepdims=True)

    @pl.when(i == 0)
    def _():
        accmax[...] = vmax
        accsum[...] = vsum

    @pl.when(i > 0)
    def _():
        accmax[...] = jnp.maximum(accmax[...], vmax)
        accsum[...] = accsum[...] + vsum

    @pl.when(i == NBLK - 1)
    def _():
        x2max = accmax[...]
        x2mean = accsum[...] * (1.0 / K2)
        ft = (jnp.dot(x1_ref[...], w1_ref[...])
              + jnp.dot(x2max, w2_ref[...])
              + jnp.dot(x2mean, w3_ref[...]) + fcb_ref[...])
        ft = jnp.maximum(ft, 0.0)
        for j, o_ref in enumerate((o1_ref, o2_ref, o3_ref)):
            hh = jnp.maximum(
                jnp.dot(ft, hw_ref[j]) + hb_ref[j], 0.0)
            o = jnp.dot(hh, ow_ref[j]) + ob_ref[j, 0, 0]
            if j == 0:
                o = 1.0 / (1.0 + jnp.exp(-o))
            o_ref[...] = o


def _tail(h2, g2c, m2c, x1, w1, w2, w3, fcb, hw, hb, ow, ob):
    return pl.pallas_call(
        _tail_kernel,
        grid=(NBLK,),
        in_specs=[pl.BlockSpec((BLK, 128), lambda i: (i, 0)),
                  pl.BlockSpec((BLK, 1), lambda i: (i, 0)),
                  pl.BlockSpec((BLK, 1), lambda i: (i, 0)),
                  pl.BlockSpec((1, 128), lambda i: (0, 0)),
                  pl.BlockSpec((128, 64), lambda i: (0, 0)),
                  pl.BlockSpec((128, 64), lambda i: (0, 0)),
                  pl.BlockSpec((128, 64), lambda i: (0, 0)),
                  pl.BlockSpec((1, 64), lambda i: (0, 0)),
                  pl.BlockSpec((3, 64, 128), lambda i: (0, 0, 0)),
                  pl.BlockSpec((3, 1, 128), lambda i: (0, 0, 0)),
                  pl.BlockSpec((3, 128, 1), lambda i: (0, 0, 0)),
                  pl.BlockSpec((3, 1, 1), lambda i: (0, 0, 0))],
        out_specs=[pl.BlockSpec((1, 1), lambda i: (0, 0))] * 3,
        out_shape=[jax.ShapeDtypeStruct((1, 1), jnp.float32)] * 3,
        scratch_shapes=[pltpu.VMEM((1, 128), jnp.float32),
                        pltpu.VMEM((1, 128), jnp.float32)],
    )(h2, g2c, m2c, x1, w1, w2, w3, fcb, hw, hb, ow, ob)


def kernel(x, edge_index, batch, edge_attr, pos, params):
    p = params
    row = edge_index[0]
    col = edge_index[1]
    xp = jnp.pad(x, ((0, NP - N), (0, 0)))
    zeros8 = jnp.zeros((NP, 8), jnp.float32)

    y1 = _project(xp, p['w1a'])
    agg1 = _segsum8_sc(y1, row, col, zeros8)
    h1, r1 = _gin_mlp(65.0, y1, agg1, p['b1a'][None, :], p['w1b'],
                      p['b1b'][None, :], p['p1'][:, None])
    s1 = jax.nn.sigmoid(r1 / jnp.linalg.norm(p['p1']))
    s1r = s1.reshape(SROW, 128)

    g1r, m1r = _select1(s1r)
    g1c = g1r.reshape(NP, 1)
    m1c = m1r.reshape(NP, 1)
    y2, x1 = _pool_project(h1, g1c, m1c, p['w2a'], 1.0 / K1)

    agg2 = _segsum8_sc(y2, row, col, zeros8)
    w2b = jnp.pad(p['w2b'], ((0, 0), (0, 28)))
    b2b = jnp.pad(p['b2b'], (0, 28))
    p2 = jnp.pad(p['p2'], (0, 28))
    h2, r2 = _gin_mlp(101.0, y2, agg2, p['b2a'][None, :], w2b,
                      b2b[None, :], p2[:, None])
    s2 = jax.nn.sigmoid(r2 / jnp.linalg.norm(p['p2']))
    s2r = s2.reshape(SROW, 128)

    g2r, m2r = _select2(s2r, s1r, m1r)
    g2c = g2r.reshape(NP, 1)
    m2c = m2r.reshape(NP, 1)

    fcw = p['fcw']
    w1 = fcw[0:128]
    w2 = jnp.pad(fcw[128:228], ((0, 28), (0, 0)))
    w3 = jnp.pad(fcw[228:328], ((0, 28), (0, 0)))
    hw = jnp.stack([jnp.pad(p['h%dw1' % j], ((0, 0), (0, 116)))
                    for j in range(3)])
    hb = jnp.stack([jnp.pad(p['h%db1' % j], (0, 116))[None, :]
                    for j in range(3)])
    ow = jnp.stack([jnp.pad(p['h%dw2' % j], ((0, 116), (0, 0)))
                    for j in range(3)])
    ob = jnp.stack([p['h%db2' % j][None, :] for j in range(3)])

    o1, o2, o3 = _tail(h2, g2c, m2c, x1, w1, w2, w3, p['fcb'][None, :],
                       hw, hb, ow, ob)
    return (o1, o2, o3)
